# baseline (speedup 1.0000x reference)
import functools

import jax
import jax.numpy as jnp
from jax import lax
from jax.experimental import pallas as pl
from jax.experimental.pallas import tpu as pltpu
from jax.experimental.pallas import tpu_sc as plsc

_F = 26
_D = 16
_B = 16384
_NC = 2
_NS = 16
_NW = _NC * _NS
_BPW = _B // _NW
_CH = 128
_NCH = _BPW // _CH


def _emb_body(*refs):
    feats = refs[:_F]
    tables = refs[_F:2 * _F]
    out = refs[2 * _F]
    idx_v, rows_v, sem_idx, sem_g, sem_w = refs[2 * _F + 1:]

    wid = lax.axis_index("s") * _NC + lax.axis_index("c")
    base = wid * _BPW
    row0 = wid * _NCH

    idx_copies = [
        pltpu.make_async_copy(
            feats[i].at[pl.ds(row0, _NCH)], idx_v.at[i], sem_idx)
        for i in range(_F)
    ]
    for c in idx_copies:
        c.start()
    for c in idx_copies:
        c.wait()

    def gather_descs(i, p):
        return [
            pltpu.make_async_copy(
                tables[i].at[idx_v.at[i, c]],
                rows_v.at[p, pl.ds(c * _CH, _CH)],
                sem_g.at[p])
            for c in range(_NCH)
        ]

    def write_desc(i, p):
        return pltpu.make_async_copy(
            rows_v.at[p],
            out.at[pl.ds(base, _BPW), pl.ds(i * _D, _D)],
            sem_w.at[p])

    for i in range(_F):
        p = i % 2
        if i >= 2:
            write_desc(i - 2, p).wait()
        for g in gather_descs(i, p):
            g.start()
        if i >= 1:
            q = 1 - p
            for g in gather_descs(i - 1, q):
                g.wait()
            write_desc(i - 1, q).start()

    p_last = (_F - 1) % 2
    for g in gather_descs(_F - 1, p_last):
        g.wait()
    write_desc(_F - 1, p_last).start()
    write_desc(_F - 2, 1 - p_last).wait()
    write_desc(_F - 1, p_last).wait()


@functools.partial(
    pl.kernel,
    mesh=plsc.VectorSubcoreMesh(core_axis_name="c", subcore_axis_name="s"),
    out_type=jax.ShapeDtypeStruct((_B, _F * _D), jnp.float32),
    compiler_params=pltpu.CompilerParams(use_tc_tiling_on_sc=False),
    scratch_types=[
        pltpu.VMEM((_F, _NCH, _CH), jnp.int32),
        pltpu.VMEM((2, _BPW, _D), jnp.float32),
        pltpu.SemaphoreType.DMA,
        pltpu.SemaphoreType.DMA((2,)),
        pltpu.SemaphoreType.DMA((2,)),
    ],
)
def _emb_kernel(*refs):
    _emb_body(*refs)


def kernel(feat_0, feat_1, feat_2, feat_3, feat_4, feat_5, feat_6, feat_7,
           feat_8, feat_9, feat_10, feat_11, feat_12, feat_13, feat_14,
           feat_15, feat_16, feat_17, feat_18, feat_19, feat_20, feat_21,
           feat_22, feat_23, feat_24, feat_25,
           W_0, W_1, W_2, W_3, W_4, W_5, W_6, W_7, W_8, W_9, W_10, W_11,
           W_12, W_13, W_14, W_15, W_16, W_17, W_18, W_19, W_20, W_21,
           W_22, W_23, W_24, W_25):
    feats = [
        feat_0, feat_1, feat_2, feat_3, feat_4, feat_5, feat_6, feat_7,
        feat_8, feat_9, feat_10, feat_11, feat_12, feat_13, feat_14,
        feat_15, feat_16, feat_17, feat_18, feat_19, feat_20, feat_21,
        feat_22, feat_23, feat_24, feat_25,
    ]
    tables = [
        W_0, W_1, W_2, W_3, W_4, W_5, W_6, W_7, W_8, W_9, W_10, W_11,
        W_12, W_13, W_14, W_15, W_16, W_17, W_18, W_19, W_20, W_21,
        W_22, W_23, W_24, W_25,
    ]
    feats2d = [f.reshape(_B // _CH, _CH) for f in feats]
    out = _emb_kernel(*feats2d, *tables)
    return out.reshape(_B, _F, _D)
